# baseline (speedup 1.0000x reference)
import functools

import jax
import jax.numpy as jnp
from jax import lax
from jax.experimental import pallas as pl
from jax.experimental.pallas import tpu as pltpu
from jax.experimental.pallas import tpu_sc as plsc

NETS = 100000
PINS = 500000
NBX = 256
NBY = 256
XL, XH, YL, YH = 0.0, 1.0, 0.0, 1.0
BSX = (XH - XL) / NBX
BSY = (YH - YL) / NBY
UNIT_HCAP = 1.5625
UNIT_VCAP = 1.45
SH = 1.0 / (BSX * BSY * UNIT_HCAP)
SV = 1.0 / (BSX * BSY * UNIT_VCAP)

NTILES = 32
NPT = 3136
NETS_PAD = NTILES * NPT
NGROUPS = NPT // 16
SBUF = 3152
SLEN = (NTILES - 1) * NPT + SBUF
CHUNK = 2048
ROWS = CHUNK // 128
FP_PAD = 503808

NB = 1024
GRID = NETS_PAD // NB

_F32 = jnp.float32
_I32 = jnp.int32


def _sc_bbox_body(starts_hbm, ends_hbm, fp_hbm, pp_hbm,
                  oxmin, oxmax, oymin, oymax,
                  sbuf, ebuf, fpbuf, ixbuf, iybuf, pxbuf, pybuf,
                  ixbuf2, iybuf2, pxbuf2, pybuf2,
                  axmin, axmax, aymin, aymax, sem, sem2):
    nc = 2
    wid = lax.axis_index("s") * nc + lax.axis_index("c")
    nlo = pl.multiple_of(wid * NPT, 8)
    pltpu.sync_copy(starts_hbm.at[pl.ds(nlo, SBUF)], sbuf)
    pltpu.sync_copy(ends_hbm.at[pl.ds(nlo, SBUF)], ebuf)

    zeros16 = jnp.zeros((16,), _I32)
    lo_init = jnp.full((16,), 2.0, _F32)
    hi_init = jnp.full((16,), -1.0, _F32)

    def init_g(g, carry):
        sl = pl.ds(g * 16, 16)
        axmin[sl] = lo_init
        axmax[sl] = hi_init
        aymin[sl] = lo_init
        aymax[sl] = hi_init
        return carry
    lax.fori_loop(0, NGROUPS, init_g, 0)

    plo = jnp.max(plsc.load_gather(sbuf, [zeros16]))
    phi = jnp.max(plsc.load_gather(ebuf, [jnp.full((16,), NPT - 1, _I32)]))
    c0 = pl.multiple_of((plo >> 10) << 10, 1024)
    nch = (phi - c0 + (CHUNK - 1)) >> 11

    def _pre(k, ixb, iyb, pxb, pyb, sm):
        p0 = pl.multiple_of(c0 + k * CHUNK, 1024)
        pltpu.sync_copy(fp_hbm.at[pl.ds(p0, CHUNK)], fpbuf)
        for i in range(CHUNK // 16):
            sl = pl.ds(i * 16, 16)
            v = fpbuf[sl]
            v2 = v + v
            ixb[sl] = v2
            iyb[sl] = v2 + 1
        for j in range(ROWS):
            sl = pl.ds(j * 128, 128)
            pltpu.make_async_copy(pp_hbm.at[ixb.at[sl]], pxb.at[sl], sm).start()
            pltpu.make_async_copy(pp_hbm.at[iyb.at[sl]], pyb.at[sl], sm).start()

    def _drain(ixb, iyb, pxb, pyb, sm):
        for j in range(ROWS):
            sl = pl.ds(j * 128, 128)
            pltpu.make_async_copy(pp_hbm.at[ixb.at[sl]], pxb.at[sl], sm).wait()
            pltpu.make_async_copy(pp_hbm.at[iyb.at[sl]], pyb.at[sl], sm).wait()

    def _proc(k, pxb, pyb, g0):
        p0 = pl.multiple_of(c0 + k * CHUNK, 1024)
        p1 = p0 + CHUNK

        def sfirst_of(g):
            idx = jnp.minimum(g, NGROUPS - 1) * 16
            return jnp.max(plsc.load_gather(sbuf, [jnp.full((16,), idx, _I32)]))

        def g_cond(c):
            g, sfirst, gnext = c
            return (g < NGROUPS) & (sfirst < p1)

        def g_body(c):
            g, sfirst, gnext = c
            n0 = g * 16
            sl = pl.ds(n0, 16)
            s_vec = sbuf[sl]
            e_vec = ebuf[sl]
            cur0 = jnp.maximum(s_vec, p0)
            lim = jnp.minimum(e_vec, p1)
            maxlen = jnp.max(jnp.maximum(lim - cur0, 0))

            @pl.when(maxlen > 0)
            def _():
                def pin_body(t, pcarry):
                    cur, xmn, xmx, ymn, ymx = pcarry
                    pxs = []
                    pys = []
                    for u in range(4):
                        cc = cur + u
                        mask = cc < lim
                        p = jnp.clip(cc - p0, 0, CHUNK - 1)
                        px = plsc.load_gather(pxb, [p], mask=mask)
                        py = plsc.load_gather(pyb, [p], mask=mask)
                        pxs.append((jnp.where(mask, px, 2.0),
                                    jnp.where(mask, px, -1.0)))
                        pys.append((jnp.where(mask, py, 2.0),
                                    jnp.where(mask, py, -1.0)))
                    a0 = jnp.minimum(pxs[0][0], pxs[1][0])
                    a1 = jnp.minimum(pxs[2][0], pxs[3][0])
                    b0 = jnp.maximum(pxs[0][1], pxs[1][1])
                    b1 = jnp.maximum(pxs[2][1], pxs[3][1])
                    c0_ = jnp.minimum(pys[0][0], pys[1][0])
                    c1 = jnp.minimum(pys[2][0], pys[3][0])
                    d0 = jnp.maximum(pys[0][1], pys[1][1])
                    d1 = jnp.maximum(pys[2][1], pys[3][1])
                    xmn = jnp.minimum(xmn, jnp.minimum(a0, a1))
                    xmx = jnp.maximum(xmx, jnp.maximum(b0, b1))
                    ymn = jnp.minimum(ymn, jnp.minimum(c0_, c1))
                    ymx = jnp.maximum(ymx, jnp.maximum(d0, d1))
                    return (cur + 4, xmn, xmx, ymn, ymx)

                nit = (maxlen + 3) >> 2
                init = (cur0, axmin[sl], axmax[sl], aymin[sl], aymax[sl])
                _, xmn, xmx, ymn, ymx = lax.fori_loop(0, nit, pin_body, init)
                axmin[sl] = xmn
                axmax[sl] = xmx
                aymin[sl] = ymn
                aymax[sl] = ymx

            e_last = jnp.max(e_vec)
            gnext = jnp.where((gnext < 0) & (e_last > p1), g, gnext)
            return (g + 1, sfirst_of(g + 1), gnext)

        g_end, _, gnext = lax.while_loop(
            g_cond, g_body, (g0, sfirst_of(g0), jnp.int32(-1)))
        return jnp.where(gnext < 0, g_end, gnext)

    bufa = (ixbuf, iybuf, pxbuf, pybuf, sem)
    bufb = (ixbuf2, iybuf2, pxbuf2, pybuf2, sem2)

    @pl.when(nch > 0)
    def _():
        _pre(0, *bufa)

    def chunk2_body(kk, g0):
        k0 = 2 * kk

        @pl.when(k0 + 1 < nch)
        def _():
            _pre(k0 + 1, *bufb)
        _drain(*bufa)
        g0 = _proc(k0, pxbuf, pybuf, g0)

        def odd(g):
            @pl.when(k0 + 2 < nch)
            def _():
                _pre(k0 + 2, *bufa)
            _drain(*bufb)
            return _proc(k0 + 1, pxbuf2, pybuf2, g)

        return lax.cond(k0 + 1 < nch, odd, lambda g: g, g0)

    lax.fori_loop(0, (nch + 1) >> 1, chunk2_body, jnp.int32(0))

    pltpu.sync_copy(axmin, oxmin.at[wid])
    pltpu.sync_copy(axmax, oxmax.at[wid])
    pltpu.sync_copy(aymin, oymin.at[wid])
    pltpu.sync_copy(aymax, oymax.at[wid])


def _sc_bbox(starts_hbm, ends_hbm, fp_hbm, pp_hbm):
    mesh = plsc.VectorSubcoreMesh(core_axis_name="c", subcore_axis_name="s")
    f = functools.partial(
        pl.kernel,
        out_type=[jax.ShapeDtypeStruct((NTILES, NPT), _F32)] * 4,
        mesh=mesh,
        scratch_types=[
            pltpu.VMEM((SBUF,), _I32),
            pltpu.VMEM((SBUF,), _I32),
            pltpu.VMEM((CHUNK,), _I32),
            pltpu.VMEM((CHUNK,), _I32),
            pltpu.VMEM((CHUNK,), _I32),
            pltpu.VMEM((CHUNK,), _F32),
            pltpu.VMEM((CHUNK,), _F32),
            pltpu.VMEM((CHUNK,), _I32),
            pltpu.VMEM((CHUNK,), _I32),
            pltpu.VMEM((CHUNK,), _F32),
            pltpu.VMEM((CHUNK,), _F32),
            pltpu.VMEM((NPT,), _F32),
            pltpu.VMEM((NPT,), _F32),
            pltpu.VMEM((NPT,), _F32),
            pltpu.VMEM((NPT,), _F32),
            pltpu.SemaphoreType.DMA,
            pltpu.SemaphoreType.DMA,
        ],
        compiler_params=pltpu.CompilerParams(needs_layout_passes=False),
    )(_sc_bbox_body)
    return f(starts_hbm, ends_hbm, fp_hbm, pp_hbm)


def _tc_rudy_body(xmin_ref, xmax_ref, ymin_ref, ymax_ref, s_ref, e_ref, w_ref,
                  route_ref, h_ref, v_ref):
    i = pl.program_id(0)
    xmin = jnp.maximum(xmin_ref[0, 0, :].reshape(1, NB), XL) * (1.0 / BSX)
    xmax = jnp.minimum(xmax_ref[0, 0, :].reshape(1, NB), XH) * (1.0 / BSX)
    ymin_r = jnp.maximum(ymin_ref[0, 0, :].reshape(1, NB), YL) * (1.0 / BSY)
    ymax_r = jnp.minimum(ymax_ref[0, 0, :].reshape(1, NB), YH) * (1.0 / BSY)
    deg = (e_ref[0, 0, :] - s_ref[0, 0, :]).reshape(1, NB)
    wts = w_ref[0, 0, :].reshape(1, NB)
    w = (xmax - xmin) * BSX
    h = (ymax_r - ymin_r) * BSY
    valid = (deg >= 2) & (w > 0.0) & (h > 0.0)
    wt = jnp.where(valid, wts, 0.0)
    ch = wt / jnp.where(valid, h, 1.0)
    cv = wt / jnp.where(valid, w, 1.0)
    binx = lax.broadcasted_iota(_I32, (NBX, NB), 0).astype(_F32)
    biny = lax.broadcasted_iota(_I32, (NBY, NB), 0).astype(_F32)
    oxt = jnp.clip(jnp.minimum(xmax, binx + 1.0) - jnp.maximum(xmin, binx), 0.0, None)
    oyt = jnp.clip(jnp.minimum(ymax_r, biny + 1.0) - jnp.maximum(ymin_r, biny), 0.0, None)

    @pl.when(i == 0)
    def _():
        h_ref[...] = jnp.zeros((NBX, NBY), _F32)
        v_ref[...] = jnp.zeros((NBX, NBY), _F32)

    dn = (((1,), (1,)), ((), ()))
    h_ref[...] += lax.dot_general(oxt * ch, oyt, dn, preferred_element_type=_F32)
    v_ref[...] += lax.dot_general(oxt * cv, oyt, dn, preferred_element_type=_F32)

    @pl.when(i == GRID - 1)
    def _():
        hh = h_ref[...] * (SH * BSX * BSY)
        vv = v_ref[...] * (SV * BSX * BSY)
        h_ref[...] = hh
        v_ref[...] = vv
        route_ref[...] = jnp.maximum(jnp.abs(hh), jnp.abs(vv))


def _tc_rudy(xmin, xmax, ymin, ymax, s_tc, e_tc, w_tc):
    blk = pl.BlockSpec((1, 1, NB), lambda i: (i, 0, 0))
    out_blk = pl.BlockSpec((NBX, NBY), lambda i: (0, 0))
    out_shape = [jax.ShapeDtypeStruct((NBX, NBY), _F32)] * 3
    return pl.pallas_call(
        _tc_rudy_body,
        grid=(GRID,),
        in_specs=[blk] * 7,
        out_specs=[out_blk] * 3,
        out_shape=out_shape,
        compiler_params=pltpu.CompilerParams(
            dimension_semantics=("arbitrary",)),
    )(xmin, xmax, ymin, ymax, s_tc, e_tc, w_tc)


def kernel(pin_pos, netpin_start, flat_netpin, net_weights):
    netpin_start = netpin_start.astype(_I32)
    tail = jnp.full((SLEN + 8 - (NETS + 1),), PINS, _I32)
    starts_full = jnp.concatenate([netpin_start, tail])
    starts_hbm = starts_full[:SLEN]
    ends_hbm = starts_full[1:SLEN + 1]
    fp_pad = jnp.concatenate(
        [flat_netpin.astype(_I32), jnp.zeros((FP_PAD - PINS,), _I32)])

    xmin, xmax, ymin, ymax = _sc_bbox(starts_hbm, ends_hbm, fp_pad, pin_pos)

    shp = (GRID, 1, NB)
    xmin = xmin.reshape(shp)
    xmax = xmax.reshape(shp)
    ymin = ymin.reshape(shp)
    ymax = ymax.reshape(shp)
    s_tc = starts_full[:NETS_PAD].reshape(shp)
    e_tc = starts_full[1:NETS_PAD + 1].reshape(shp)
    w_tc = jnp.concatenate(
        [net_weights, jnp.zeros((NETS_PAD - NETS,), _F32)]).reshape(shp)

    route, hmap, vmap = _tc_rudy(xmin, xmax, ymin, ymax, s_tc, e_tc, w_tc)
    return route, hmap, vmap


# speedup vs baseline: 1.2113x; 1.2113x over previous
import functools

import jax
import jax.numpy as jnp
from jax import lax
from jax.experimental import pallas as pl
from jax.experimental.pallas import tpu as pltpu
from jax.experimental.pallas import tpu_sc as plsc

NETS = 100000
PINS = 500000
NBX = 256
NBY = 256
XL, XH, YL, YH = 0.0, 1.0, 0.0, 1.0
BSX = (XH - XL) / NBX
BSY = (YH - YL) / NBY
UNIT_HCAP = 1.5625
UNIT_VCAP = 1.45
SH = 1.0 / (BSX * BSY * UNIT_HCAP)
SV = 1.0 / (BSX * BSY * UNIT_VCAP)

NTILES = 32
NPT = 3136
NETS_PAD = NTILES * NPT
NGROUPS = NPT // 16
SBUF = 3152
SLEN = (NTILES - 1) * NPT + SBUF
CHUNK = 2048
ROWS = CHUNK // 128
WLAST = PINS - CHUNK

NB = 2048
GRID = NETS_PAD // NB

_F32 = jnp.float32
_I32 = jnp.int32


def _sc_bbox_body(starts_hbm, ends_hbm, fp_hbm, pp_hbm,
                  oxmin, oxmax, oymin, oymax,
                  sbuf, ebuf, fpbuf, ixbuf, iybuf, pxbuf, pybuf,
                  ixbuf2, iybuf2, pxbuf2, pybuf2,
                  axmin, axmax, aymin, aymax, sem, sem2):
    nc = 2
    wid = lax.axis_index("s") * nc + lax.axis_index("c")
    nlo = pl.multiple_of(wid * NPT, 8)
    pltpu.sync_copy(starts_hbm.at[pl.ds(nlo, SBUF)], sbuf)
    pltpu.sync_copy(ends_hbm.at[pl.ds(nlo, SBUF)], ebuf)

    zeros16 = jnp.zeros((16,), _I32)
    lo_init = jnp.full((16,), 2.0, _F32)
    hi_init = jnp.full((16,), -1.0, _F32)

    def init_g(g, carry):
        sl = pl.ds(g * 16, 16)
        axmin[sl] = lo_init
        axmax[sl] = hi_init
        aymin[sl] = lo_init
        aymax[sl] = hi_init
        return carry
    lax.fori_loop(0, NGROUPS, init_g, 0)

    plo = jnp.max(plsc.load_gather(sbuf, [zeros16]))
    phi = jnp.max(plsc.load_gather(ebuf, [jnp.full((16,), NPT - 1, _I32)]))
    c0 = pl.multiple_of((plo >> 3) << 3, 8)
    nch = (phi - c0 + (CHUNK - 1)) >> 11

    def _wbase(k):
        return pl.multiple_of(jnp.minimum(c0 + k * CHUNK, WLAST), 8)

    def _pre(k, ixb, iyb, pxb, pyb, sm):
        p0 = _wbase(k)
        pltpu.sync_copy(fp_hbm.at[pl.ds(p0, CHUNK)], fpbuf)
        for i in range(CHUNK // 16):
            sl = pl.ds(i * 16, 16)
            v = fpbuf[sl]
            v2 = v + v
            ixb[sl] = v2
            iyb[sl] = v2 + 1
        for j in range(ROWS):
            sl = pl.ds(j * 128, 128)
            pltpu.make_async_copy(pp_hbm.at[ixb.at[sl]], pxb.at[sl], sm).start()
            pltpu.make_async_copy(pp_hbm.at[iyb.at[sl]], pyb.at[sl], sm).start()

    def _drain(ixb, iyb, pxb, pyb, sm):
        for j in range(ROWS):
            sl = pl.ds(j * 128, 128)
            pltpu.make_async_copy(pp_hbm.at[ixb.at[sl]], pxb.at[sl], sm).wait()
            pltpu.make_async_copy(pp_hbm.at[iyb.at[sl]], pyb.at[sl], sm).wait()

    def _proc(k, pxb, pyb, g0):
        p0 = _wbase(k)
        p1 = p0 + CHUNK

        def sfirst_of(g):
            idx = jnp.minimum(g, NGROUPS - 1) * 16
            return jnp.max(plsc.load_gather(sbuf, [jnp.full((16,), idx, _I32)]))

        def g_cond(c):
            g, sfirst, gnext = c
            return (g < NGROUPS) & (sfirst < p1)

        def g_body(c):
            g, sfirst, gnext = c
            n0 = g * 16
            sl = pl.ds(n0, 16)
            s_vec = sbuf[sl]
            e_vec = ebuf[sl]
            cur0 = jnp.maximum(s_vec, p0)
            lim = jnp.minimum(e_vec, p1)
            maxlen = jnp.max(jnp.maximum(lim - cur0, 0))

            @pl.when(maxlen > 0)
            def _():
                def pin_body(t, pcarry):
                    cur, xmn, xmx, ymn, ymx = pcarry
                    pxs = []
                    pys = []
                    for u in range(4):
                        cc = cur + u
                        mask = cc < lim
                        p = jnp.clip(cc - p0, 0, CHUNK - 1)
                        px = plsc.load_gather(pxb, [p], mask=mask)
                        py = plsc.load_gather(pyb, [p], mask=mask)
                        pxs.append((jnp.where(mask, px, 2.0),
                                    jnp.where(mask, px, -1.0)))
                        pys.append((jnp.where(mask, py, 2.0),
                                    jnp.where(mask, py, -1.0)))
                    a0 = jnp.minimum(pxs[0][0], pxs[1][0])
                    a1 = jnp.minimum(pxs[2][0], pxs[3][0])
                    b0 = jnp.maximum(pxs[0][1], pxs[1][1])
                    b1 = jnp.maximum(pxs[2][1], pxs[3][1])
                    c0_ = jnp.minimum(pys[0][0], pys[1][0])
                    c1 = jnp.minimum(pys[2][0], pys[3][0])
                    d0 = jnp.maximum(pys[0][1], pys[1][1])
                    d1 = jnp.maximum(pys[2][1], pys[3][1])
                    xmn = jnp.minimum(xmn, jnp.minimum(a0, a1))
                    xmx = jnp.maximum(xmx, jnp.maximum(b0, b1))
                    ymn = jnp.minimum(ymn, jnp.minimum(c0_, c1))
                    ymx = jnp.maximum(ymx, jnp.maximum(d0, d1))
                    return (cur + 4, xmn, xmx, ymn, ymx)

                nit = (maxlen + 3) >> 2
                init = (cur0, axmin[sl], axmax[sl], aymin[sl], aymax[sl])
                _, xmn, xmx, ymn, ymx = lax.fori_loop(0, nit, pin_body, init)
                axmin[sl] = xmn
                axmax[sl] = xmx
                aymin[sl] = ymn
                aymax[sl] = ymx

            e_last = jnp.max(e_vec)
            gnext = jnp.where((gnext < 0) & (e_last > p1), g, gnext)
            return (g + 1, sfirst_of(g + 1), gnext)

        g_end, _, gnext = lax.while_loop(
            g_cond, g_body, (g0, sfirst_of(g0), jnp.int32(-1)))
        return jnp.where(gnext < 0, g_end, gnext)

    bufa = (ixbuf, iybuf, pxbuf, pybuf, sem)
    bufb = (ixbuf2, iybuf2, pxbuf2, pybuf2, sem2)

    @pl.when(nch > 0)
    def _():
        _pre(0, *bufa)

    def chunk2_body(kk, g0):
        k0 = 2 * kk

        @pl.when(k0 + 1 < nch)
        def _():
            _pre(k0 + 1, *bufb)
        _drain(*bufa)
        g0 = _proc(k0, pxbuf, pybuf, g0)

        def odd(g):
            @pl.when(k0 + 2 < nch)
            def _():
                _pre(k0 + 2, *bufa)
            _drain(*bufb)
            return _proc(k0 + 1, pxbuf2, pybuf2, g)

        return lax.cond(k0 + 1 < nch, odd, lambda g: g, g0)

    lax.fori_loop(0, (nch + 1) >> 1, chunk2_body, jnp.int32(0))

    pltpu.sync_copy(axmin, oxmin.at[wid])
    pltpu.sync_copy(axmax, oxmax.at[wid])
    pltpu.sync_copy(aymin, oymin.at[wid])
    pltpu.sync_copy(aymax, oymax.at[wid])


def _sc_bbox(starts_hbm, ends_hbm, fp_hbm, pp_hbm):
    mesh = plsc.VectorSubcoreMesh(core_axis_name="c", subcore_axis_name="s")
    f = functools.partial(
        pl.kernel,
        out_type=[jax.ShapeDtypeStruct((NTILES, NPT), _F32)] * 4,
        mesh=mesh,
        scratch_types=[
            pltpu.VMEM((SBUF,), _I32),
            pltpu.VMEM((SBUF,), _I32),
            pltpu.VMEM((CHUNK,), _I32),
            pltpu.VMEM((CHUNK,), _I32),
            pltpu.VMEM((CHUNK,), _I32),
            pltpu.VMEM((CHUNK,), _F32),
            pltpu.VMEM((CHUNK,), _F32),
            pltpu.VMEM((CHUNK,), _I32),
            pltpu.VMEM((CHUNK,), _I32),
            pltpu.VMEM((CHUNK,), _F32),
            pltpu.VMEM((CHUNK,), _F32),
            pltpu.VMEM((NPT,), _F32),
            pltpu.VMEM((NPT,), _F32),
            pltpu.VMEM((NPT,), _F32),
            pltpu.VMEM((NPT,), _F32),
            pltpu.SemaphoreType.DMA,
            pltpu.SemaphoreType.DMA,
        ],
        compiler_params=pltpu.CompilerParams(needs_layout_passes=False),
    )(_sc_bbox_body)
    return f(starts_hbm, ends_hbm, fp_hbm, pp_hbm)


def _tc_rudy_body(xmin_ref, xmax_ref, ymin_ref, ymax_ref, s_ref, e_ref, w_ref,
                  route_ref, h_ref, v_ref):
    i = pl.program_id(0)
    xmin = jnp.maximum(xmin_ref[0, 0, :].reshape(1, NB), XL) * (1.0 / BSX)
    xmax = jnp.minimum(xmax_ref[0, 0, :].reshape(1, NB), XH) * (1.0 / BSX)
    ymin_r = jnp.maximum(ymin_ref[0, 0, :].reshape(1, NB), YL) * (1.0 / BSY)
    ymax_r = jnp.minimum(ymax_ref[0, 0, :].reshape(1, NB), YH) * (1.0 / BSY)
    deg = (e_ref[0, 0, :] - s_ref[0, 0, :]).reshape(1, NB)
    wts = w_ref[0, 0, :].reshape(1, NB)
    w = (xmax - xmin) * BSX
    h = (ymax_r - ymin_r) * BSY
    valid = (deg >= 2) & (w > 0.0) & (h > 0.0)
    wt = jnp.where(valid, wts, 0.0)
    ch = wt / jnp.where(valid, h, 1.0)
    cv = wt / jnp.where(valid, w, 1.0)
    binx = lax.broadcasted_iota(_I32, (NBX, NB), 0).astype(_F32)
    biny = lax.broadcasted_iota(_I32, (NBY, NB), 0).astype(_F32)
    oxt = jnp.clip(jnp.minimum(xmax, binx + 1.0) - jnp.maximum(xmin, binx), 0.0, None)
    oyt = jnp.clip(jnp.minimum(ymax_r, biny + 1.0) - jnp.maximum(ymin_r, biny), 0.0, None)

    @pl.when(i == 0)
    def _():
        h_ref[...] = jnp.zeros((NBX, NBY), _F32)
        v_ref[...] = jnp.zeros((NBX, NBY), _F32)

    dn = (((1,), (1,)), ((), ()))
    h_ref[...] += lax.dot_general(oxt * ch, oyt, dn, preferred_element_type=_F32)
    v_ref[...] += lax.dot_general(oxt * cv, oyt, dn, preferred_element_type=_F32)

    @pl.when(i == GRID - 1)
    def _():
        hh = h_ref[...] * (SH * BSX * BSY)
        vv = v_ref[...] * (SV * BSX * BSY)
        h_ref[...] = hh
        v_ref[...] = vv
        route_ref[...] = jnp.maximum(jnp.abs(hh), jnp.abs(vv))


def _tc_rudy(xmin, xmax, ymin, ymax, s_tc, e_tc, w_tc):
    blk = pl.BlockSpec((1, 1, NB), lambda i: (i, 0, 0))
    out_blk = pl.BlockSpec((NBX, NBY), lambda i: (0, 0))
    out_shape = [jax.ShapeDtypeStruct((NBX, NBY), _F32)] * 3
    return pl.pallas_call(
        _tc_rudy_body,
        grid=(GRID,),
        in_specs=[blk] * 7,
        out_specs=[out_blk] * 3,
        out_shape=out_shape,
        compiler_params=pltpu.CompilerParams(
            dimension_semantics=("arbitrary",)),
    )(xmin, xmax, ymin, ymax, s_tc, e_tc, w_tc)


def kernel(pin_pos, netpin_start, flat_netpin, net_weights):
    netpin_start = netpin_start.astype(_I32)
    tail = jnp.full((SLEN + 8 - (NETS + 1),), PINS, _I32)
    starts_full = jnp.concatenate([netpin_start, tail])
    starts_hbm = starts_full[:SLEN]
    ends_hbm = starts_full[1:SLEN + 1]
    xmin, xmax, ymin, ymax = _sc_bbox(
        starts_hbm, ends_hbm, flat_netpin.astype(_I32), pin_pos)

    shp = (GRID, 1, NB)
    xmin = xmin.reshape(shp)
    xmax = xmax.reshape(shp)
    ymin = ymin.reshape(shp)
    ymax = ymax.reshape(shp)
    s_tc = starts_full[:NETS_PAD].reshape(shp)
    e_tc = starts_full[1:NETS_PAD + 1].reshape(shp)
    w_tc = jnp.concatenate(
        [net_weights, jnp.zeros((NETS_PAD - NETS,), _F32)]).reshape(shp)

    route, hmap, vmap = _tc_rudy(xmin, xmax, ymin, ymax, s_tc, e_tc, w_tc)
    return route, hmap, vmap
